# baseline (speedup 1.0000x reference)
import functools

import jax
import jax.numpy as jnp
from jax import lax
from jax.experimental import pallas as pl
from jax.experimental.pallas import tpu as pltpu
from jax.experimental.pallas import tpu_sc as plsc

NUM_REL = 16
DIM = 64
N_EDGES = 800000

_info = plsc.get_sparse_core_info()
_NC, _NS = _info.num_cores, _info.num_subcores
_NW = _NC * _NS
_N_TILES = N_EDGES // 128
_T_BASE = _N_TILES // _NW
_T_EXTRA = _N_TILES % _NW
_TPC = 6
_ECH = 128 * _TPC
_N_CHUNKS = -(-(_T_BASE + 1) // _TPC)
_N_PAIRS = (_N_CHUNKS - 1) // 2


def _make_sc_kernel():
    mesh = plsc.VectorSubcoreMesh(core_axis_name="c", subcore_axis_name="s")

    @functools.partial(
        pl.kernel,
        mesh=mesh,
        compiler_params=pltpu.CompilerParams(needs_layout_passes=False),
        out_type=jax.ShapeDtypeStruct((DIM, N_EDGES), jnp.float32),
        scratch_types=[
            pltpu.VMEM((2, _ECH), jnp.int32),
            pltpu.VMEM((DIM, _ECH), jnp.float32),
            pltpu.VMEM((DIM, _ECH), jnp.float32),
            pltpu.VMEM((NUM_REL * DIM,), jnp.float32),
            pltpu.SemaphoreType.DMA,
            pltpu.SemaphoreType.DMA,
            pltpu.SemaphoreType.DMA,
            pltpu.SemaphoreType.DMA,
        ],
    )
    def k(idx_hbm, table_hbm, out_hbm, idx_v, buf0, buf1, table_v,
          si0, si1, sw0, sw1):
        wid = lax.axis_index("s") * _NC + lax.axis_index("c")
        ts = wid * _T_BASE + jnp.minimum(wid, _T_EXTRA)
        nt = _T_BASE + jnp.where(wid < _T_EXTRA, 1, 0)
        bufs = (buf0, buf1)
        si = (si0, si1)
        sw = (sw0, sw1)

        pltpu.sync_copy(table_hbm, table_v)

        def chunk_e0(kk):
            return 128 * (ts + jnp.minimum(kk * _TPC, nt - _TPC))

        def stage_idx(kk, b):
            pltpu.async_copy(
                idx_hbm.at[pl.ds(chunk_e0(kk), _ECH)], idx_v.at[b], si[b]
            )

        def wait_idx(kk, b):
            pltpu.make_async_copy(
                idx_hbm.at[pl.ds(chunk_e0(kk), _ECH)], idx_v.at[b], si[b]
            ).wait()

        def compute_chunk(b):
            buf = bufs[b]

            def group(eg, carry):
                e0 = eg * 16
                idxv = idx_v[b, pl.ds(e0, 16)]
                for c0 in range(0, DIM, 8):
                    vals = [
                        plsc.load_gather(table_v, [idxv + (c0 + u) * 16])
                        for u in range(8)
                    ]
                    for u in range(8):
                        buf[c0 + u, pl.ds(e0, 16)] = vals[u]
                return carry

            lax.fori_loop(0, _ECH // 16, group, 0)

        def writeback(kk, b):
            e0 = chunk_e0(kk)
            pltpu.async_copy(
                bufs[b], out_hbm.at[:, pl.ds(e0, _ECH)], sw[b]
            )

        def drain(kk, b):
            e0 = chunk_e0(kk)
            pltpu.make_async_copy(
                bufs[b], out_hbm.at[:, pl.ds(e0, _ECH)], sw[b]
            ).wait()

        stage_idx(0, 0)
        wait_idx(0, 0)
        stage_idx(1, 1)
        compute_chunk(0)
        writeback(0, 0)

        def pair(it, carry):
            k1 = 1 + 2 * it
            k0 = 2 + 2 * it
            wait_idx(k1, 1)
            stage_idx(k0, 0)

            @pl.when(it > 0)
            def _():
                drain(k1 - 2, 1)

            compute_chunk(1)
            writeback(k1, 1)

            wait_idx(k0, 0)

            @pl.when(k0 + 1 < _N_CHUNKS)
            def _():
                stage_idx(k0 + 1, 1)

            drain(k0 - 2, 0)
            compute_chunk(0)
            writeback(k0, 0)
            return carry

        lax.fori_loop(0, _N_PAIRS, pair, 0)
        drain(_N_CHUNKS - 2, 1)
        drain(_N_CHUNKS - 1, 0)

    return k


_sc_kernel = _make_sc_kernel()


def kernel(relation_indices, W):
    idx = relation_indices.astype(jnp.int32)
    table_t = jnp.reshape(jnp.transpose(W), (-1,))
    out_t = _sc_kernel(idx, table_t)
    return jnp.transpose(out_t)


# speedup vs baseline: 1.0138x; 1.0138x over previous
import functools

import jax
import jax.numpy as jnp
from jax import lax
from jax.experimental import pallas as pl
from jax.experimental.pallas import tpu as pltpu
from jax.experimental.pallas import tpu_sc as plsc

NUM_REL = 16
DIM = 64
N_EDGES = 800000

_info = plsc.get_sparse_core_info()
_NC, _NS = _info.num_cores, _info.num_subcores
_NW = _NC * _NS
_N_TILES = N_EDGES // 128
_T_BASE = _N_TILES // _NW
_T_EXTRA = _N_TILES % _NW
_TPC = 4
_ECH = 128 * _TPC
_N_CHUNKS = -(-(_T_BASE + 1) // _TPC)
_N_PAIRS = (_N_CHUNKS - 1) // 2


def _make_sc_kernel():
    mesh = plsc.VectorSubcoreMesh(core_axis_name="c", subcore_axis_name="s")

    @functools.partial(
        pl.kernel,
        mesh=mesh,
        compiler_params=pltpu.CompilerParams(needs_layout_passes=False),
        out_type=jax.ShapeDtypeStruct((DIM, N_EDGES), jnp.float32),
        scratch_types=[
            pltpu.VMEM((2, _ECH), jnp.int32),
            pltpu.VMEM((DIM, _ECH), jnp.float32),
            pltpu.VMEM((DIM, _ECH), jnp.float32),
            pltpu.VMEM((NUM_REL * DIM,), jnp.float32),
            pltpu.SemaphoreType.DMA,
            pltpu.SemaphoreType.DMA,
            pltpu.SemaphoreType.DMA,
            pltpu.SemaphoreType.DMA,
        ],
    )
    def k(idx_hbm, table_hbm, out_hbm, idx_v, buf0, buf1, table_v,
          si0, si1, sw0, sw1):
        wid = lax.axis_index("s") * _NC + lax.axis_index("c")
        ts = wid * _T_BASE + jnp.minimum(wid, _T_EXTRA)
        nt = _T_BASE + jnp.where(wid < _T_EXTRA, 1, 0)
        bufs = (buf0, buf1)
        si = (si0, si1)
        sw = (sw0, sw1)

        pltpu.sync_copy(table_hbm, table_v)

        def chunk_e0(kk):
            return 128 * (ts + jnp.minimum(kk * _TPC, nt - _TPC))

        def stage_idx(kk, b):
            pltpu.async_copy(
                idx_hbm.at[pl.ds(chunk_e0(kk), _ECH)], idx_v.at[b], si[b]
            )

        def wait_idx(kk, b):
            pltpu.make_async_copy(
                idx_hbm.at[pl.ds(chunk_e0(kk), _ECH)], idx_v.at[b], si[b]
            ).wait()

        def compute_chunk(b):
            buf = bufs[b]

            def group(eg, carry):
                e0 = eg * 16
                idxv = idx_v[b, pl.ds(e0, 16)]
                for c0 in range(0, DIM, 8):
                    vals = [
                        plsc.load_gather(table_v, [idxv + (c0 + u) * 16])
                        for u in range(8)
                    ]
                    for u in range(8):
                        buf[c0 + u, pl.ds(e0, 16)] = vals[u]
                return carry

            lax.fori_loop(0, _ECH // 16, group, 0)

        def writeback(kk, b):
            e0 = chunk_e0(kk)
            pltpu.async_copy(
                bufs[b], out_hbm.at[:, pl.ds(e0, _ECH)], sw[b]
            )

        def drain(kk, b):
            e0 = chunk_e0(kk)
            pltpu.make_async_copy(
                bufs[b], out_hbm.at[:, pl.ds(e0, _ECH)], sw[b]
            ).wait()

        stage_idx(0, 0)
        wait_idx(0, 0)
        stage_idx(1, 1)
        compute_chunk(0)
        writeback(0, 0)

        def pair(it, carry):
            k1 = 1 + 2 * it
            k0 = 2 + 2 * it
            wait_idx(k1, 1)
            stage_idx(k0, 0)

            @pl.when(it > 0)
            def _():
                drain(k1 - 2, 1)

            compute_chunk(1)
            writeback(k1, 1)

            wait_idx(k0, 0)

            @pl.when(k0 + 1 < _N_CHUNKS)
            def _():
                stage_idx(k0 + 1, 1)

            drain(k0 - 2, 0)
            compute_chunk(0)
            writeback(k0, 0)
            return carry

        lax.fori_loop(0, _N_PAIRS, pair, 0)
        drain(_N_CHUNKS - 2, 1)
        drain(_N_CHUNKS - 1, 0)

    return k


_sc_kernel = _make_sc_kernel()


def kernel(relation_indices, W):
    idx = relation_indices.astype(jnp.int32)
    table_t = jnp.reshape(jnp.transpose(W), (-1,))
    out_t = _sc_kernel(idx, table_t)
    return jnp.transpose(out_t)
